# baseline (speedup 1.0000x reference)
import functools
import math

import jax
import jax.numpy as jnp
from jax import lax
from jax.experimental import pallas as pl
from jax.experimental.pallas import tpu as pltpu
from jax.experimental.pallas import tpu_sc as plsc

VOCAB = 100000
EMB = 128
SCALE = math.sqrt(EMB)

NC = 2
NS = 16
NW = NC * NS

CHUNK = 128


def _scale_body(emb_ref, out_ref):
    out_ref[...] = emb_ref[...] * SCALE


def _scale_table(embeddings):
    rows = embeddings.shape[0]
    block = 5000
    grid = rows // block
    return pl.pallas_call(
        _scale_body,
        grid=(grid,),
        in_specs=[pl.BlockSpec((block, EMB), lambda i: (i, 0))],
        out_specs=pl.BlockSpec((block, EMB), lambda i: (i, 0)),
        out_shape=jax.ShapeDtypeStruct((rows, EMB), jnp.float32),
    )(embeddings)


NBUF = 4


def _make_gather(n_tokens):
    per_w = n_tokens // NW
    n_stages = per_w // CHUNK
    mesh = plsc.VectorSubcoreMesh(core_axis_name="c", subcore_axis_name="s")

    @functools.partial(
        pl.kernel,
        mesh=mesh,
        out_type=jax.ShapeDtypeStruct((n_tokens, EMB), jnp.float32),
        scratch_types=[
            pltpu.VMEM((n_stages, CHUNK), jnp.int32),
        ] + [pltpu.VMEM((CHUNK, EMB), jnp.float32)] * NBUF
          + [pltpu.SemaphoreType.DMA] * (2 * NBUF),
    )
    def gather_kernel(tok_hbm, table_hbm, out_hbm, idx_v, *bufs):
        rows = bufs[:NBUF]
        gsem = bufs[NBUF:2 * NBUF]
        wsem = bufs[2 * NBUF:]
        wid = lax.axis_index("s") * NC + lax.axis_index("c")
        row_base = wid * per_w

        pltpu.sync_copy(tok_hbm.at[pl.ds(wid * n_stages, n_stages)], idx_v)

        def g_start(s, b):
            pltpu.async_copy(table_hbm.at[idx_v.at[s]], rows[b], gsem[b])

        def g_wait(b):
            pltpu.make_async_copy(table_hbm.at[idx_v.at[0]], rows[b],
                                  gsem[b]).wait()

        def w_start(s, b):
            pltpu.async_copy(rows[b],
                             out_hbm.at[pl.ds(row_base + s * CHUNK, CHUNK)],
                             wsem[b])

        def w_wait(b):
            pltpu.make_async_copy(rows[b], out_hbm.at[pl.ds(row_base, CHUNK)],
                                  wsem[b]).wait()

        for s0 in range(NBUF - 1):
            g_start(s0, s0)

        def scale_rows(b):
            buf = rows[b]

            def srow(r, carry):
                for rr in range(2):
                    for k in range(8):
                        sl = (r * 2 + rr, pl.ds(k * 16, 16))
                        buf[sl] = buf[sl] * SCALE
                return carry

            lax.fori_loop(0, CHUNK // 2, srow, 0)

        def phase(s, b):
            g_wait(b)
            nxt = (b + NBUF - 1) % NBUF

            @pl.when(s + NBUF - 1 < n_stages)
            def _():
                @pl.when(s >= 1)
                def _():
                    w_wait(nxt)
                g_start(s + NBUF - 1, nxt)

            scale_rows(b)
            w_start(s, b)

        def body(i, carry):
            s = i * NBUF
            for b in range(NBUF):
                phase(s + b, b)
            return carry

        lax.fori_loop(0, n_stages // NBUF, body, 0)
        for b in range(NBUF):
            w_wait(b)

    return gather_kernel


def kernel(tokens, embeddings):
    b, t = tokens.shape
    flat = tokens.reshape(b * t // CHUNK, CHUNK).astype(jnp.int32)
    out = _make_gather(b * t)(flat, embeddings)
    return out.reshape(b, t, EMB)


# speedup vs baseline: 1.0014x; 1.0014x over previous
import functools
import math

import jax
import jax.numpy as jnp
from jax import lax
from jax.experimental import pallas as pl
from jax.experimental.pallas import tpu as pltpu
from jax.experimental.pallas import tpu_sc as plsc

EMB = 128
SCALE = math.sqrt(EMB)

NC = 2
NS = 16
NW = NC * NS

ICH = 128
STG = 2
GCH = STG * ICH
NBUF = 3


def _make_gather(n_tokens):
    per_w = n_tokens // NW
    n_stages = per_w // GCH
    n_body = (n_stages // NBUF) * NBUF
    mesh = plsc.VectorSubcoreMesh(core_axis_name="c", subcore_axis_name="s")

    @functools.partial(
        pl.kernel,
        mesh=mesh,
        out_type=jax.ShapeDtypeStruct((n_tokens, EMB), jnp.float32),
        scratch_types=[
            pltpu.VMEM((n_stages * STG, ICH), jnp.int32),
        ] + [pltpu.VMEM((GCH, EMB), jnp.float32)] * NBUF
          + [pltpu.SemaphoreType.DMA] * (2 * NBUF),
    )
    def gather_kernel(tok_hbm, table_hbm, out_hbm, idx_v, *bufs):
        rows = bufs[:NBUF]
        gsem = bufs[NBUF:2 * NBUF]
        wsem = bufs[2 * NBUF:]
        wid = lax.axis_index("s") * NC + lax.axis_index("c")
        row_base = wid * per_w

        pltpu.sync_copy(tok_hbm.at[wid], idx_v)

        def g_start(s, b):
            for j in range(STG):
                pltpu.async_copy(table_hbm.at[idx_v.at[s * STG + j]],
                                 rows[b].at[pl.ds(j * ICH, ICH)], gsem[b])

        def g_wait(b):
            for j in range(STG):
                pltpu.make_async_copy(table_hbm.at[idx_v.at[0]],
                                      rows[b].at[pl.ds(j * ICH, ICH)],
                                      gsem[b]).wait()

        def w_start(s, b):
            pltpu.async_copy(rows[b],
                             out_hbm.at[pl.ds(row_base + s * GCH, GCH)],
                             wsem[b])

        def w_wait(b):
            pltpu.make_async_copy(rows[b], out_hbm.at[pl.ds(row_base, GCH)],
                                  wsem[b]).wait()

        def scale_rows(b):
            buf = rows[b]

            def srow(r, carry):
                for rr in range(2):
                    for k in range(8):
                        sl = (r * 2 + rr, pl.ds(k * 16, 16))
                        buf[sl] = buf[sl] * SCALE
                return carry

            lax.fori_loop(0, GCH // 2, srow, 0)

        for s0 in range(NBUF - 1):
            g_start(s0, s0)

        def phase(s, b):
            g_wait(b)
            nxt = (b + NBUF - 1) % NBUF

            @pl.when(s + NBUF - 1 < n_stages)
            def _():
                @pl.when(s >= 1)
                def _():
                    w_wait(nxt)
                g_start(s + NBUF - 1, nxt)

            scale_rows(b)
            w_start(s, b)

        def body(i, carry):
            s = i * NBUF
            for b in range(NBUF):
                phase(s + b, b)
            return carry

        lax.fori_loop(0, n_stages // NBUF, body, 0)
        for s in range(n_body, n_stages):
            phase(s, s % NBUF)
        for b in range(NBUF):
            w_wait(b)

    return gather_kernel


def kernel(tokens, embeddings):
    b, t = tokens.shape
    flat = tokens.reshape(NW, b * t // (NW * ICH), ICH).astype(jnp.int32)
    out = _make_gather(b * t)(flat, embeddings)
    return out.reshape(b, t, EMB)
